# Initial kernel scaffold; baseline (speedup 1.0000x reference)
import jax
import jax.numpy as jnp
from jax.experimental import pallas as pl


def kernel(feats, coords):
    raise NotImplementedError("write your pallas kernel here")


# speedup vs baseline: 2.6311x; 2.6311x over previous
import functools

import jax
import jax.numpy as jnp
from jax import lax
from jax.experimental import pallas as pl
from jax.experimental.pallas import tpu as pltpu
from jax.experimental.pallas import tpu_sc as plsc

_D = 64
_FAC = 2
_DC = _D // _FAC
_M = _DC ** 3
_R = _M * _FAC ** 3

_NC = 2
_NS = 16
_NW = _NC * _NS
_BLK = 128
_PSLICE = _R // _NS


def _sc_scatter(xs, ys, zs, feats):
    n, c = feats.shape
    nblk = -(-n // (_NW * _BLK))
    chunk = nblk * _BLK

    mesh = plsc.VectorSubcoreMesh(core_axis_name="c", subcore_axis_name="s")

    @functools.partial(
        pl.kernel,
        out_type=[
            jax.ShapeDtypeStruct((_R, c), jnp.float32),
            jax.ShapeDtypeStruct((_NC, _R), jnp.float32),
        ],
        mesh=mesh,
        scratch_types=[
            pltpu.VMEM((chunk,), jnp.int32),
            pltpu.VMEM((chunk,), jnp.int32),
            pltpu.VMEM((chunk,), jnp.int32),
            pltpu.VMEM((_BLK,), jnp.int32),
            pltpu.VMEM((_BLK, c), jnp.float32),
            pltpu.VMEM((_BLK,), jnp.float32),
            pltpu.VMEM((_PSLICE,), jnp.float32),
            pltpu.VMEM_SHARED((_R,), jnp.float32),
        ],
    )
    def run(xs_hbm, ys_hbm, zs_hbm, feats_hbm, scat_hbm, pres_hbm,
            xs_v, ys_v, zs_v, idx_v, rows_v, ones_v, zeros_v, pres_sh):
        cid = lax.axis_index("c")
        sid = lax.axis_index("s")
        wid = sid * _NC + cid
        base = jnp.minimum(wid * chunk, n - chunk)

        for j in range(_BLK // 16):
            ones_v[pl.ds(j * 16, 16)] = jnp.full((16,), 1.0, jnp.float32)

        def zstep(i, carry):
            zeros_v[pl.ds(i * 16, 16)] = jnp.zeros((16,), jnp.float32)
            return carry

        lax.fori_loop(0, _PSLICE // 16, zstep, 0)
        pltpu.sync_copy(zeros_v, pres_sh.at[pl.ds(sid * _PSLICE, _PSLICE)])
        plsc.subcore_barrier()

        pltpu.sync_copy(xs_hbm.at[pl.ds(base, chunk)], xs_v)
        pltpu.sync_copy(ys_hbm.at[pl.ds(base, chunk)], ys_v)
        pltpu.sync_copy(zs_hbm.at[pl.ds(base, chunk)], zs_v)

        def blk_step(blk, carry):
            for t in range(_BLK // 16):
                off = blk * _BLK + t * 16
                xv = xs_v[pl.ds(off, 16)]
                yv = ys_v[pl.ds(off, 16)]
                zv = zs_v[pl.ds(off, 16)]
                cc = ((xv >> 1) * _DC + (yv >> 1)) * _DC + (zv >> 1)
                bb = (yv & 1) + 2 * (xv & 1) + 4 * (zv & 1)
                idx_v[pl.ds(t * 16, 16)] = cc * 8 + bb
            pltpu.sync_copy(ones_v, pres_sh.at[idx_v], add=True)
            pltpu.sync_copy(feats_hbm.at[pl.ds(base + blk * _BLK, _BLK)], rows_v)
            pltpu.sync_copy(rows_v, scat_hbm.at[idx_v])
            return carry

        lax.fori_loop(0, nblk, blk_step, 0)

        plsc.subcore_barrier()
        pltpu.sync_copy(
            pres_sh.at[pl.ds(sid * _PSLICE, _PSLICE)],
            pres_hbm.at[cid, pl.ds(sid * _PSLICE, _PSLICE)],
        )

    return run(xs, ys, zs, feats)


_TC_ROWS = 2048
_TC_CELLS = _TC_ROWS // 8


def _tc_body(scat_ref, pa_ref, pb_ref, out_ref):
    c = scat_ref.shape[1]
    p = pa_ref[...] + pb_ref[...]
    p3 = p.reshape(_TC_CELLS, 8, 1)
    present = (p3 > 0.5).astype(jnp.float32)
    s3 = scat_ref[...].reshape(_TC_CELLS, 8, c)
    masked = jnp.where(present > 0.5, s3, 0.0)
    sums = masked.sum(axis=1)
    counts = present.sum(axis=1)
    avg = sums / jnp.maximum(counts, 1.0)
    out3 = jnp.where(present > 0.5, s3, avg[:, None, :])
    out_ref[...] = out3.reshape(_TC_ROWS, c)


def _tc_combine(scat, pa, pb):
    c = scat.shape[1]
    grid = _R // _TC_ROWS
    return pl.pallas_call(
        _tc_body,
        grid=(grid,),
        in_specs=[
            pl.BlockSpec((_TC_ROWS, c), lambda i: (i, 0)),
            pl.BlockSpec((_TC_ROWS, 1), lambda i: (i, 0)),
            pl.BlockSpec((_TC_ROWS, 1), lambda i: (i, 0)),
        ],
        out_specs=pl.BlockSpec((_TC_ROWS, c), lambda i: (i, 0)),
        out_shape=jax.ShapeDtypeStruct((_R, c), jnp.float32),
    )(scat, pa, pb)


def kernel(feats, coords):
    n, c = feats.shape
    xs = coords[:, 0].astype(jnp.int32)
    ys = coords[:, 1].astype(jnp.int32)
    zs = coords[:, 2].astype(jnp.int32)
    scat, pres = _sc_scatter(xs, ys, zs, feats)
    pa = pres[0].reshape(_R, 1)
    pb = pres[1].reshape(_R, 1)
    out = _tc_combine(scat, pa, pb)
    return out.reshape(_M, _FAC ** 3 * c)


# speedup vs baseline: 3.7242x; 1.4154x over previous
import functools

import jax
import jax.numpy as jnp
from jax import lax
from jax.experimental import pallas as pl
from jax.experimental.pallas import tpu as pltpu
from jax.experimental.pallas import tpu_sc as plsc

_D = 64
_FAC = 2
_DC = _D // _FAC
_M = _DC ** 3
_R = _M * _FAC ** 3

_NC = 2
_NS = 16
_NW = _NC * _NS
_BLK = 128
_PSLICE = _R // _NS


def _sc_scatter(xs, ys, zs, feats):
    n, c = feats.shape
    nblk = -(-n // (_NW * _BLK))
    chunk = nblk * _BLK

    mesh = plsc.VectorSubcoreMesh(core_axis_name="c", subcore_axis_name="s")

    @functools.partial(
        pl.kernel,
        out_type=[
            jax.ShapeDtypeStruct((_R, c), jnp.float32),
            jax.ShapeDtypeStruct((_NC, _R), jnp.float32),
        ],
        mesh=mesh,
        scratch_types=[
            pltpu.VMEM((chunk,), jnp.int32),
            pltpu.VMEM((chunk,), jnp.int32),
            pltpu.VMEM((chunk,), jnp.int32),
            pltpu.VMEM((_BLK,), jnp.int32),
            pltpu.VMEM((_BLK, c), jnp.float32),
            pltpu.VMEM((_BLK,), jnp.float32),
            pltpu.VMEM((_PSLICE,), jnp.float32),
            pltpu.VMEM_SHARED((_R,), jnp.float32),
        ],
    )
    def run(xs_hbm, ys_hbm, zs_hbm, feats_hbm, scat_hbm, pres_hbm,
            xs_v, ys_v, zs_v, idx_v, rows_v, ones_v, zeros_v, pres_sh):
        cid = lax.axis_index("c")
        sid = lax.axis_index("s")
        wid = sid * _NC + cid
        base = jnp.minimum(wid * chunk, n - chunk)

        for j in range(_BLK // 16):
            ones_v[pl.ds(j * 16, 16)] = jnp.full((16,), 1.0, jnp.float32)

        def zstep(i, carry):
            zeros_v[pl.ds(i * 16, 16)] = jnp.zeros((16,), jnp.float32)
            return carry

        lax.fori_loop(0, _PSLICE // 16, zstep, 0)
        pltpu.sync_copy(zeros_v, pres_sh.at[pl.ds(sid * _PSLICE, _PSLICE)])
        plsc.subcore_barrier()

        pltpu.sync_copy(xs_hbm.at[pl.ds(base, chunk)], xs_v)
        pltpu.sync_copy(ys_hbm.at[pl.ds(base, chunk)], ys_v)
        pltpu.sync_copy(zs_hbm.at[pl.ds(base, chunk)], zs_v)

        def blk_step(blk, carry):
            for t in range(_BLK // 16):
                off = blk * _BLK + t * 16
                xv = xs_v[pl.ds(off, 16)]
                yv = ys_v[pl.ds(off, 16)]
                zv = zs_v[pl.ds(off, 16)]
                cc = ((xv >> 1) * _DC + (yv >> 1)) * _DC + (zv >> 1)
                bb = (yv & 1) + 2 * (xv & 1) + 4 * (zv & 1)
                idx_v[pl.ds(t * 16, 16)] = cc * 8 + bb
            pltpu.sync_copy(ones_v, pres_sh.at[idx_v], add=True)
            pltpu.sync_copy(feats_hbm.at[pl.ds(base + blk * _BLK, _BLK)], rows_v)
            pltpu.sync_copy(rows_v, scat_hbm.at[idx_v])
            return carry

        lax.fori_loop(0, nblk, blk_step, 0)

        plsc.subcore_barrier()
        pltpu.sync_copy(
            pres_sh.at[pl.ds(sid * _PSLICE, _PSLICE)],
            pres_hbm.at[cid, pl.ds(sid * _PSLICE, _PSLICE)],
        )

    return run(xs, ys, zs, feats)


_TC_ROWS = 2048
_TC_CELLS = _TC_ROWS // 8


def _tc_body(scat_ref, pres_ref, out_ref):
    c = scat_ref.shape[1]
    lanes = pres_ref.shape[2]
    sub = _TC_ROWS // lanes
    pr = pres_ref[...]
    plane = pr[0] + pr[1]
    spread = jnp.broadcast_to(plane[:, None, :], (sub, lanes, lanes))
    spread = spread.reshape(_TC_ROWS, lanes)
    lane_id = lax.broadcasted_iota(jnp.int32, (_TC_ROWS, lanes), 1)
    row_mod = lax.broadcasted_iota(jnp.int32, (_TC_ROWS, lanes), 0) % lanes
    p = jnp.where(lane_id == row_mod, spread, 0.0).sum(axis=1, keepdims=True)
    p3 = p.reshape(_TC_CELLS, 8, 1)
    present = (p3 > 0.5).astype(jnp.float32)
    s3 = scat_ref[...].reshape(_TC_CELLS, 8, c)
    masked = jnp.where(present > 0.5, s3, 0.0)
    sums = masked.sum(axis=1)
    counts = present.sum(axis=1)
    avg = sums / jnp.maximum(counts, 1.0)
    out3 = jnp.where(present > 0.5, s3, avg[:, None, :])
    out_ref[...] = out3.reshape(_TC_ROWS, c)


def _tc_combine(scat, pres):
    c = scat.shape[1]
    grid = _R // _TC_ROWS
    pres3 = pres.reshape(2, _R // 128, 128)
    return pl.pallas_call(
        _tc_body,
        grid=(grid,),
        in_specs=[
            pl.BlockSpec((_TC_ROWS, c), lambda i: (i, 0)),
            pl.BlockSpec((2, _TC_ROWS // 128, 128), lambda i: (0, i, 0)),
        ],
        out_specs=pl.BlockSpec((_TC_ROWS, c), lambda i: (i, 0)),
        out_shape=jax.ShapeDtypeStruct((_R, c), jnp.float32),
    )(scat, pres3)


def kernel(feats, coords):
    n, c = feats.shape
    xs = coords[:, 0].astype(jnp.int32)
    ys = coords[:, 1].astype(jnp.int32)
    zs = coords[:, 2].astype(jnp.int32)
    scat, pres = _sc_scatter(xs, ys, zs, feats)
    out = _tc_combine(scat, pres)
    return out.reshape(_M, _FAC ** 3 * c)


# speedup vs baseline: 5.4011x; 1.4503x over previous
import functools

import jax
import jax.numpy as jnp
from jax import lax
from jax.experimental import pallas as pl
from jax.experimental.pallas import tpu as pltpu
from jax.experimental.pallas import tpu_sc as plsc

_D = 64
_FAC = 2
_DC = _D // _FAC
_M = _DC ** 3
_R = _M * _FAC ** 3

_NC = 2
_NS = 16
_NW = _NC * _NS
_BLK = 128
_PSLICE = _R // _NS


def _sc_scatter(xs, ys, zs, feats):
    n, c = feats.shape
    nblk = -(-n // (_NW * _BLK))
    chunk = nblk * _BLK

    mesh = plsc.VectorSubcoreMesh(core_axis_name="c", subcore_axis_name="s")

    @functools.partial(
        pl.kernel,
        out_type=[
            jax.ShapeDtypeStruct((_R, c), jnp.float32),
            jax.ShapeDtypeStruct((_NC, _R), jnp.float32),
        ],
        mesh=mesh,
        scratch_types=[
            pltpu.VMEM((chunk,), jnp.int32),
            pltpu.VMEM((chunk,), jnp.int32),
            pltpu.VMEM((chunk,), jnp.int32),
            pltpu.VMEM((_BLK,), jnp.int32),
            pltpu.VMEM((_BLK, c), jnp.float32),
            pltpu.VMEM((_BLK,), jnp.float32),
            pltpu.VMEM((_PSLICE,), jnp.float32),
            pltpu.VMEM_SHARED((_R,), jnp.float32),
        ],
    )
    def run(xs_hbm, ys_hbm, zs_hbm, feats_hbm, scat_hbm, pres_hbm,
            xs_v, ys_v, zs_v, idx_v, rows_v, ones_v, zeros_v, pres_sh):
        cid = lax.axis_index("c")
        sid = lax.axis_index("s")
        wid = sid * _NC + cid
        base = jnp.minimum(wid * chunk, n - chunk)

        for j in range(_BLK // 16):
            ones_v[pl.ds(j * 16, 16)] = jnp.full((16,), 1.0, jnp.float32)

        def zstep(i, carry):
            zeros_v[pl.ds(i * 16, 16)] = jnp.zeros((16,), jnp.float32)
            return carry

        lax.fori_loop(0, _PSLICE // 16, zstep, 0)
        pltpu.sync_copy(zeros_v, pres_sh.at[pl.ds(sid * _PSLICE, _PSLICE)])
        plsc.subcore_barrier()

        pltpu.sync_copy(xs_hbm.at[pl.ds(base, chunk)], xs_v)
        pltpu.sync_copy(ys_hbm.at[pl.ds(base, chunk)], ys_v)
        pltpu.sync_copy(zs_hbm.at[pl.ds(base, chunk)], zs_v)

        def blk_step(blk, carry):
            for t in range(_BLK // 16):
                off = blk * _BLK + t * 16
                xv = xs_v[pl.ds(off, 16)]
                yv = ys_v[pl.ds(off, 16)]
                zv = zs_v[pl.ds(off, 16)]
                cc = ((xv >> 1) * _DC + (yv >> 1)) * _DC + (zv >> 1)
                bb = (yv & 1) + 2 * (xv & 1) + 4 * (zv & 1)
                idx_v[pl.ds(t * 16, 16)] = cc * 8 + bb
            pltpu.sync_copy(ones_v, pres_sh.at[idx_v], add=True)
            pltpu.sync_copy(feats_hbm.at[pl.ds(base + blk * _BLK, _BLK)], rows_v)
            pltpu.sync_copy(rows_v, scat_hbm.at[idx_v])
            return carry

        lax.fori_loop(0, nblk, blk_step, 0)

        plsc.subcore_barrier()
        pltpu.sync_copy(
            pres_sh.at[pl.ds(sid * _PSLICE, _PSLICE)],
            pres_hbm.at[cid, pl.ds(sid * _PSLICE, _PSLICE)],
        )

    return run(xs, ys, zs, feats)


_TC_ROWS = 2048
_TC_CELLS = _TC_ROWS // 8


def _tc_body(scat_ref, pres_ref, out_ref):
    c = scat_ref.shape[1]
    lanes = pres_ref.shape[2]
    sub = _TC_ROWS // lanes
    pr = pres_ref[...]
    plane = pr[0] + pr[1]
    spread = jnp.broadcast_to(plane[:, None, :], (sub, lanes, lanes))
    spread = spread.reshape(_TC_ROWS, lanes)
    lane_id = lax.broadcasted_iota(jnp.int32, (_TC_ROWS, lanes), 1)
    row_mod = lax.broadcasted_iota(jnp.int32, (_TC_ROWS, lanes), 0) % lanes
    p = jnp.where(lane_id == row_mod, spread, 0.0).sum(axis=1, keepdims=True)
    p3 = p.reshape(_TC_CELLS, 8, 1)
    present = (p3 > 0.5).astype(jnp.float32)
    s3 = scat_ref[...].reshape(_TC_CELLS, 8, c)
    masked = jnp.where(present > 0.5, s3, 0.0)
    sums = masked.sum(axis=1)
    counts = present.sum(axis=1)
    avg = sums / jnp.maximum(counts, 1.0)
    out3 = jnp.where(present > 0.5, s3, avg[:, None, :])
    out_ref[...] = out3.reshape(_TC_CELLS, 8 * c)


def _tc_combine(scat, pres):
    c = scat.shape[1]
    grid = _R // _TC_ROWS
    pres3 = pres.reshape(2, _R // 128, 128)
    return pl.pallas_call(
        _tc_body,
        grid=(grid,),
        in_specs=[
            pl.BlockSpec((_TC_ROWS, c), lambda i: (i, 0)),
            pl.BlockSpec((2, _TC_ROWS // 128, 128), lambda i: (0, i, 0)),
        ],
        out_specs=pl.BlockSpec((_TC_CELLS, 8 * c), lambda i: (i, 0)),
        out_shape=jax.ShapeDtypeStruct((_M, 8 * c), jnp.float32),
    )(scat, pres3)


def kernel(feats, coords):
    n, c = feats.shape
    xs = coords[:, 0].astype(jnp.int32)
    ys = coords[:, 1].astype(jnp.int32)
    zs = coords[:, 2].astype(jnp.int32)
    scat, pres = _sc_scatter(xs, ys, zs, feats)
    return _tc_combine(scat, pres)


# speedup vs baseline: 5.5804x; 1.0332x over previous
import functools

import jax
import jax.numpy as jnp
from jax import lax
from jax.experimental import pallas as pl
from jax.experimental.pallas import tpu as pltpu
from jax.experimental.pallas import tpu_sc as plsc

_D = 64
_FAC = 2
_DC = _D // _FAC
_M = _DC ** 3
_R = _M * _FAC ** 3

_NC = 2
_NS = 16
_NW = _NC * _NS
_BLK = 128
_PSLICE = _R // _NS


def _sc_scatter(xs, ys, zs, feats):
    n, c = feats.shape
    nblk = -(-n // (_NW * _BLK))
    chunk = nblk * _BLK

    mesh = plsc.VectorSubcoreMesh(core_axis_name="c", subcore_axis_name="s")

    @functools.partial(
        pl.kernel,
        out_type=[
            jax.ShapeDtypeStruct((_R, c), jnp.float32),
            jax.ShapeDtypeStruct((_NC, _R), jnp.float32),
        ],
        mesh=mesh,
        scratch_types=[
            pltpu.VMEM((chunk,), jnp.int32),
            pltpu.VMEM((chunk,), jnp.int32),
            pltpu.VMEM((chunk,), jnp.int32),
            pltpu.VMEM((nblk, _BLK), jnp.int32),
            pltpu.VMEM((_BLK, c), jnp.float32),
            pltpu.VMEM((_BLK, c), jnp.float32),
            pltpu.VMEM((_BLK,), jnp.float32),
            pltpu.VMEM((_PSLICE,), jnp.float32),
            pltpu.VMEM_SHARED((_R,), jnp.float32),
            pltpu.SemaphoreType.DMA,
            pltpu.SemaphoreType.DMA,
            pltpu.SemaphoreType.DMA,
            pltpu.SemaphoreType.DMA,
            pltpu.SemaphoreType.DMA,
        ],
    )
    def run(xs_hbm, ys_hbm, zs_hbm, feats_hbm, scat_hbm, pres_hbm,
            xs_v, ys_v, zs_v, idx_v, rows0_v, rows1_v, ones_v, zeros_v,
            pres_sh, sl0, sl1, ss0, ss1, sp):
        cid = lax.axis_index("c")
        sid = lax.axis_index("s")
        wid = sid * _NC + cid
        base = jnp.minimum(wid * chunk, n - chunk)
        rows = (rows0_v, rows1_v)
        sems_l = (sl0, sl1)
        sems_s = (ss0, ss1)

        for j in range(_BLK // 16):
            ones_v[pl.ds(j * 16, 16)] = jnp.full((16,), 1.0, jnp.float32)

        def zstep(i, carry):
            zeros_v[pl.ds(i * 16, 16)] = jnp.zeros((16,), jnp.float32)
            return carry

        lax.fori_loop(0, _PSLICE // 16, zstep, 0)
        pltpu.sync_copy(zeros_v, pres_sh.at[pl.ds(sid * _PSLICE, _PSLICE)])
        ld0 = pltpu.async_copy(feats_hbm.at[pl.ds(base, _BLK)], rows[0], sems_l[0])
        plsc.subcore_barrier()

        pltpu.sync_copy(xs_hbm.at[pl.ds(base, chunk)], xs_v)
        pltpu.sync_copy(ys_hbm.at[pl.ds(base, chunk)], ys_v)
        pltpu.sync_copy(zs_hbm.at[pl.ds(base, chunk)], zs_v)

        for blk in range(nblk):
            for t in range(_BLK // 16):
                off = blk * _BLK + t * 16
                xv = xs_v[pl.ds(off, 16)]
                yv = ys_v[pl.ds(off, 16)]
                zv = zs_v[pl.ds(off, 16)]
                cc = ((xv >> 1) * _DC + (yv >> 1)) * _DC + (zv >> 1)
                bb = (yv & 1) + 2 * (xv & 1) + 4 * (zv & 1)
                idx_v[blk, pl.ds(t * 16, 16)] = cc * 8 + bb

        padds = [
            pltpu.async_copy(ones_v, pres_sh.at[idx_v.at[blk]], sp, add=True)
            for blk in range(nblk)
        ]

        loads = [ld0] + [None] * (nblk - 1)
        scats = [None] * nblk
        for blk in range(nblk):
            b = blk % 2
            loads[blk].wait()
            if blk + 1 < nblk:
                if blk >= 1:
                    scats[blk - 1].wait()
                loads[blk + 1] = pltpu.async_copy(
                    feats_hbm.at[pl.ds(base + (blk + 1) * _BLK, _BLK)],
                    rows[1 - b], sems_l[1 - b])
            scats[blk] = pltpu.async_copy(
                rows[b], scat_hbm.at[idx_v.at[blk]], sems_s[b])
        scats[nblk - 1].wait()
        if nblk >= 2:
            scats[nblk - 2].wait()
        for cp in padds:
            cp.wait()

        plsc.subcore_barrier()
        pltpu.sync_copy(
            pres_sh.at[pl.ds(sid * _PSLICE, _PSLICE)],
            pres_hbm.at[cid, pl.ds(sid * _PSLICE, _PSLICE)],
        )

    return run(xs, ys, zs, feats)


_TC_ROWS = 2048
_TC_CELLS = _TC_ROWS // 8


def _tc_body(scat_ref, pres_ref, out_ref):
    c = scat_ref.shape[1]
    lanes = pres_ref.shape[2]
    sub = _TC_ROWS // lanes
    pr = pres_ref[...]
    plane = pr[0] + pr[1]
    spread = jnp.broadcast_to(plane[:, None, :], (sub, lanes, lanes))
    spread = spread.reshape(_TC_ROWS, lanes)
    lane_id = lax.broadcasted_iota(jnp.int32, (_TC_ROWS, lanes), 1)
    row_mod = lax.broadcasted_iota(jnp.int32, (_TC_ROWS, lanes), 0) % lanes
    p = jnp.where(lane_id == row_mod, spread, 0.0).sum(axis=1, keepdims=True)
    p3 = p.reshape(_TC_CELLS, 8, 1)
    present = (p3 > 0.5).astype(jnp.float32)
    s3 = scat_ref[...].reshape(_TC_CELLS, 8, c)
    masked = jnp.where(present > 0.5, s3, 0.0)
    sums = masked.sum(axis=1)
    counts = present.sum(axis=1)
    avg = sums / jnp.maximum(counts, 1.0)
    out3 = jnp.where(present > 0.5, s3, avg[:, None, :])
    out_ref[...] = out3.reshape(_TC_CELLS, 8 * c)


def _tc_combine(scat, pres):
    c = scat.shape[1]
    grid = _R // _TC_ROWS
    pres3 = pres.reshape(2, _R // 128, 128)
    return pl.pallas_call(
        _tc_body,
        grid=(grid,),
        in_specs=[
            pl.BlockSpec((_TC_ROWS, c), lambda i: (i, 0)),
            pl.BlockSpec((2, _TC_ROWS // 128, 128), lambda i: (0, i, 0)),
        ],
        out_specs=pl.BlockSpec((_TC_CELLS, 8 * c), lambda i: (i, 0)),
        out_shape=jax.ShapeDtypeStruct((_M, 8 * c), jnp.float32),
    )(scat, pres3)


def kernel(feats, coords):
    n, c = feats.shape
    xs = coords[:, 0].astype(jnp.int32)
    ys = coords[:, 1].astype(jnp.int32)
    zs = coords[:, 2].astype(jnp.int32)
    scat, pres = _sc_scatter(xs, ys, zs, feats)
    return _tc_combine(scat, pres)


# speedup vs baseline: 6.4744x; 1.1602x over previous
import functools

import jax
import jax.numpy as jnp
from jax import lax
from jax.experimental import pallas as pl
from jax.experimental.pallas import tpu as pltpu
from jax.experimental.pallas import tpu_sc as plsc

_D = 64
_FAC = 2
_DC = _D // _FAC
_M = _DC ** 3
_R = _M * _FAC ** 3

_NC = 2
_NS = 16
_NW = _NC * _NS
_BLK = 128
_PSLICE = _R // _NS


def _sc_scatter(xs, ys, zs, feats):
    n, c = feats.shape
    nblk = -(-n // (_NW * _BLK))
    chunk = nblk * _BLK

    mesh = plsc.VectorSubcoreMesh(core_axis_name="c", subcore_axis_name="s")

    @functools.partial(
        pl.kernel,
        out_type=[
            jax.ShapeDtypeStruct((_R, c), jnp.float32),
            jax.ShapeDtypeStruct((_NC, _R), jnp.float32),
        ],
        mesh=mesh,
        scratch_types=[
            pltpu.VMEM((chunk,), jnp.int32),
            pltpu.VMEM((chunk,), jnp.int32),
            pltpu.VMEM((chunk,), jnp.int32),
            pltpu.VMEM((nblk, _BLK), jnp.int32),
            pltpu.VMEM((_BLK, c), jnp.float32),
            pltpu.VMEM((_BLK, c), jnp.float32),
            pltpu.VMEM((_BLK,), jnp.float32),
            pltpu.VMEM((_PSLICE,), jnp.float32),
            pltpu.VMEM_SHARED((_R,), jnp.float32),
            pltpu.SemaphoreType.DMA,
            pltpu.SemaphoreType.DMA,
            pltpu.SemaphoreType.DMA,
            pltpu.SemaphoreType.DMA,
            pltpu.SemaphoreType.DMA,
        ],
    )
    def run(xs_hbm, ys_hbm, zs_hbm, feats_hbm, scat_hbm, pres_hbm,
            xs_v, ys_v, zs_v, idx_v, rows0_v, rows1_v, ones_v, zeros_v,
            pres_sh, sl0, sl1, ss0, ss1, sp):
        cid = lax.axis_index("c")
        sid = lax.axis_index("s")
        wid = sid * _NC + cid
        base = jnp.minimum(wid * chunk, n - chunk)
        rows = (rows0_v, rows1_v)
        sems_l = (sl0, sl1)
        sems_s = (ss0, ss1)

        for j in range(_BLK // 16):
            ones_v[pl.ds(j * 16, 16)] = jnp.full((16,), 1.0, jnp.float32)

        def zstep(i, carry):
            zeros_v[pl.ds(i * 16, 16)] = jnp.zeros((16,), jnp.float32)
            return carry

        lax.fori_loop(0, _PSLICE // 16, zstep, 0)
        pltpu.sync_copy(zeros_v, pres_sh.at[pl.ds(sid * _PSLICE, _PSLICE)])
        ld0 = pltpu.async_copy(feats_hbm.at[pl.ds(base, _BLK)], rows[0], sems_l[0])
        plsc.subcore_barrier()

        pltpu.sync_copy(xs_hbm.at[pl.ds(base, chunk)], xs_v)
        pltpu.sync_copy(ys_hbm.at[pl.ds(base, chunk)], ys_v)
        pltpu.sync_copy(zs_hbm.at[pl.ds(base, chunk)], zs_v)

        for blk in range(nblk):
            for t in range(_BLK // 16):
                off = blk * _BLK + t * 16
                xv = xs_v[pl.ds(off, 16)]
                yv = ys_v[pl.ds(off, 16)]
                zv = zs_v[pl.ds(off, 16)]
                cc = ((xv >> 1) * _DC + (yv >> 1)) * _DC + (zv >> 1)
                bb = (yv & 1) + 2 * (xv & 1) + 4 * (zv & 1)
                idx_v[blk, pl.ds(t * 16, 16)] = cc * 8 + bb

        padds = [
            pltpu.async_copy(ones_v, pres_sh.at[idx_v.at[blk]], sp, add=True)
            for blk in range(nblk)
        ]

        loads = [ld0] + [None] * (nblk - 1)
        scats = [None] * nblk
        for blk in range(nblk):
            b = blk % 2
            loads[blk].wait()
            if blk + 1 < nblk:
                if blk >= 1:
                    scats[blk - 1].wait()
                loads[blk + 1] = pltpu.async_copy(
                    feats_hbm.at[pl.ds(base + (blk + 1) * _BLK, _BLK)],
                    rows[1 - b], sems_l[1 - b])
            scats[blk] = pltpu.async_copy(
                rows[b], scat_hbm.at[idx_v.at[blk]], sems_s[b])
        scats[nblk - 1].wait()
        if nblk >= 2:
            scats[nblk - 2].wait()
        for cp in padds:
            cp.wait()

        plsc.subcore_barrier()
        pltpu.sync_copy(
            pres_sh.at[pl.ds(sid * _PSLICE, _PSLICE)],
            pres_hbm.at[cid, pl.ds(sid * _PSLICE, _PSLICE)],
        )

    return run(xs, ys, zs, feats)


_TC_ROWS = 4096
_TC_CELLS = _TC_ROWS // 8


def _tc_body(scat_ref, pres_ref, out_ref):
    c = scat_ref.shape[1]
    lanes = pres_ref.shape[2]
    sub = _TC_ROWS // lanes
    pr = pres_ref[...]
    plane = pr[0] + pr[1]
    spread = jnp.broadcast_to(plane[:, None, :], (sub, lanes, lanes))
    spread = spread.reshape(_TC_ROWS, lanes)
    lane_id = lax.broadcasted_iota(jnp.int32, (_TC_ROWS, lanes), 1)
    row_mod = lax.broadcasted_iota(jnp.int32, (_TC_ROWS, lanes), 0) % lanes
    p = jnp.where(lane_id == row_mod, spread, 0.0).sum(axis=1, keepdims=True)
    p3 = p.reshape(_TC_CELLS, 8, 1)
    pb = p3 > 0.5
    present = pb.astype(jnp.float32)
    s3 = scat_ref[...].reshape(_TC_CELLS, 8, c)
    masked = jnp.where(pb, s3, 0.0)
    sums = masked.sum(axis=1)
    counts = present.sum(axis=1)
    avg = sums / jnp.maximum(counts, 1.0)
    out3 = jnp.where(pb, s3, avg[:, None, :])
    out_ref[...] = out3.reshape(_TC_CELLS, 8 * c)


def _tc_combine(scat, pres):
    c = scat.shape[1]
    grid = _R // _TC_ROWS
    pres3 = pres.reshape(2, _R // 128, 128)
    return pl.pallas_call(
        _tc_body,
        grid=(grid,),
        in_specs=[
            pl.BlockSpec((_TC_ROWS, c), lambda i: (i, 0)),
            pl.BlockSpec((2, _TC_ROWS // 128, 128), lambda i: (0, i, 0)),
        ],
        out_specs=pl.BlockSpec((_TC_CELLS, 8 * c), lambda i: (i, 0)),
        out_shape=jax.ShapeDtypeStruct((_M, 8 * c), jnp.float32),
    )(scat, pres3)


def kernel(feats, coords):
    n, c = feats.shape
    xs = coords[:, 0].astype(jnp.int32)
    ys = coords[:, 1].astype(jnp.int32)
    zs = coords[:, 2].astype(jnp.int32)
    scat, pres = _sc_scatter(xs, ys, zs, feats)
    return _tc_combine(scat, pres)


# speedup vs baseline: 6.9275x; 1.0700x over previous
import functools

import jax
import jax.numpy as jnp
from jax import lax
from jax.experimental import pallas as pl
from jax.experimental.pallas import tpu as pltpu
from jax.experimental.pallas import tpu_sc as plsc

_D = 64
_FAC = 2
_DC = _D // _FAC
_M = _DC ** 3
_R = _M * _FAC ** 3

_NC = 2
_NS = 16
_NW = _NC * _NS
_BLK = 128
_PSLICE = _R // _NS


def _sc_scatter(xs, ys, zs, feats):
    n, c = feats.shape
    nblk = -(-n // (_NW * _BLK))
    chunk = nblk * _BLK

    mesh = plsc.VectorSubcoreMesh(core_axis_name="c", subcore_axis_name="s")

    @functools.partial(
        pl.kernel,
        out_type=[
            jax.ShapeDtypeStruct((_R, c), jnp.float32),
            jax.ShapeDtypeStruct((_NC, _R), jnp.float32),
        ],
        mesh=mesh,
        scratch_types=[
            pltpu.VMEM((chunk,), jnp.int32),
            pltpu.VMEM((chunk,), jnp.int32),
            pltpu.VMEM((chunk,), jnp.int32),
            pltpu.VMEM((nblk, _BLK), jnp.int32),
            pltpu.VMEM((_BLK, c), jnp.float32),
            pltpu.VMEM((_BLK, c), jnp.float32),
            pltpu.VMEM((_BLK,), jnp.float32),
            pltpu.VMEM((_PSLICE,), jnp.float32),
            pltpu.VMEM_SHARED((_R,), jnp.float32),
            pltpu.SemaphoreType.DMA,
            pltpu.SemaphoreType.DMA,
            pltpu.SemaphoreType.DMA,
            pltpu.SemaphoreType.DMA,
            pltpu.SemaphoreType.DMA,
        ],
    )
    def run(xs_hbm, ys_hbm, zs_hbm, feats_hbm, scat_hbm, pres_hbm,
            xs_v, ys_v, zs_v, idx_v, rows0_v, rows1_v, ones_v, zeros_v,
            pres_sh, sl0, sl1, ss0, ss1, sp):
        cid = lax.axis_index("c")
        sid = lax.axis_index("s")
        wid = sid * _NC + cid
        base = jnp.minimum(wid * chunk, n - chunk)
        rows = (rows0_v, rows1_v)
        sems_l = (sl0, sl1)
        sems_s = (ss0, ss1)

        for j in range(_BLK // 16):
            ones_v[pl.ds(j * 16, 16)] = jnp.full((16,), 1.0, jnp.float32)

        def zstep(i, carry):
            zeros_v[pl.ds(i * 16, 16)] = jnp.zeros((16,), jnp.float32)
            return carry

        lax.fori_loop(0, _PSLICE // 16, zstep, 0)
        pltpu.sync_copy(zeros_v, pres_sh.at[pl.ds(sid * _PSLICE, _PSLICE)])
        ld0 = pltpu.async_copy(feats_hbm.at[pl.ds(base, _BLK)], rows[0], sems_l[0])
        plsc.subcore_barrier()

        pltpu.sync_copy(xs_hbm.at[pl.ds(base, chunk)], xs_v)
        pltpu.sync_copy(ys_hbm.at[pl.ds(base, chunk)], ys_v)
        pltpu.sync_copy(zs_hbm.at[pl.ds(base, chunk)], zs_v)

        for blk in range(nblk):
            for t in range(_BLK // 16):
                off = blk * _BLK + t * 16
                xv = xs_v[pl.ds(off, 16)]
                yv = ys_v[pl.ds(off, 16)]
                zv = zs_v[pl.ds(off, 16)]
                cc = ((xv >> 1) * _DC + (yv >> 1)) * _DC + (zv >> 1)
                bb = (yv & 1) + 2 * (xv & 1) + 4 * (zv & 1)
                idx_v[blk, pl.ds(t * 16, 16)] = cc * 8 + bb

        padds = [
            pltpu.async_copy(ones_v, pres_sh.at[idx_v.at[blk]], sp, add=True)
            for blk in range(nblk)
        ]

        loads = [ld0] + [None] * (nblk - 1)
        scats = [None] * nblk
        for blk in range(nblk):
            b = blk % 2
            loads[blk].wait()
            if blk + 1 < nblk:
                if blk >= 1:
                    scats[blk - 1].wait()
                loads[blk + 1] = pltpu.async_copy(
                    feats_hbm.at[pl.ds(base + (blk + 1) * _BLK, _BLK)],
                    rows[1 - b], sems_l[1 - b])
            scats[blk] = pltpu.async_copy(
                rows[b], scat_hbm.at[idx_v.at[blk]], sems_s[b])
        scats[nblk - 1].wait()
        if nblk >= 2:
            scats[nblk - 2].wait()
        for cp in padds:
            cp.wait()

        plsc.subcore_barrier()
        pltpu.sync_copy(
            pres_sh.at[pl.ds(sid * _PSLICE, _PSLICE)],
            pres_hbm.at[cid, pl.ds(sid * _PSLICE, _PSLICE)],
        )

    return run(xs, ys, zs, feats)


_TC_ROWS = 8192
_TC_CELLS = _TC_ROWS // 8


def _tc_body(scat_ref, pres_ref, out_ref):
    c = scat_ref.shape[1]
    lanes = pres_ref.shape[2]
    sub = _TC_ROWS // lanes
    pr = pres_ref[...]
    plane = pr[0] + pr[1]
    spread = jnp.broadcast_to(plane[:, None, :], (sub, lanes, lanes))
    spread = spread.reshape(_TC_ROWS, lanes)
    lane_id = lax.broadcasted_iota(jnp.int32, (_TC_ROWS, lanes), 1)
    row_mod = lax.broadcasted_iota(jnp.int32, (_TC_ROWS, lanes), 0) % lanes
    p = jnp.where(lane_id == row_mod, spread, 0.0).sum(axis=1, keepdims=True)
    p3 = p.reshape(_TC_CELLS, 8, 1)
    pb = p3 > 0.5
    present = pb.astype(jnp.float32)
    s3 = scat_ref[...].reshape(_TC_CELLS, 8, c)
    masked = jnp.where(pb, s3, 0.0)
    sums = masked.sum(axis=1)
    counts = present.sum(axis=1)
    avg = sums / jnp.maximum(counts, 1.0)
    out3 = jnp.where(pb, s3, avg[:, None, :])
    out_ref[...] = out3.reshape(_TC_CELLS, 8 * c)


def _tc_combine(scat, pres):
    c = scat.shape[1]
    grid = _R // _TC_ROWS
    pres3 = pres.reshape(2, _R // 128, 128)
    return pl.pallas_call(
        _tc_body,
        grid=(grid,),
        in_specs=[
            pl.BlockSpec((_TC_ROWS, c), lambda i: (i, 0)),
            pl.BlockSpec((2, _TC_ROWS // 128, 128), lambda i: (0, i, 0)),
        ],
        out_specs=pl.BlockSpec((_TC_CELLS, 8 * c), lambda i: (i, 0)),
        out_shape=jax.ShapeDtypeStruct((_M, 8 * c), jnp.float32),
    )(scat, pres3)


def kernel(feats, coords):
    n, c = feats.shape
    xs = coords[:, 0].astype(jnp.int32)
    ys = coords[:, 1].astype(jnp.int32)
    zs = coords[:, 2].astype(jnp.int32)
    scat, pres = _sc_scatter(xs, ys, zs, feats)
    return _tc_combine(scat, pres)
